# Initial kernel scaffold; baseline (speedup 1.0000x reference)
import jax
import jax.numpy as jnp
from jax.experimental import pallas as pl


def kernel(feat, edge_index, W, b):
    raise NotImplementedError("write your pallas kernel here")


# speedup vs baseline: 3.1033x; 3.1033x over previous
import functools

import jax
import jax.numpy as jnp
from jax import lax
from jax.experimental import pallas as pl
from jax.experimental.pallas import tpu as pltpu
from jax.experimental.pallas import tpu_sc as plsc

_NC = 2
_NS = 16
_L = 16
_NW = _NC * _NS

_CHUNK = 16000


def _sc_gather_scatter(featT_flat, src, dst, n_nodes, d_in):
    cols_per_w = d_in // _NW
    words_per_w = cols_per_w * n_nodes
    n_edges = src.shape[0]
    n_chunks = n_edges // _CHUNK
    mesh = plsc.VectorSubcoreMesh(core_axis_name="c", subcore_axis_name="s")

    @functools.partial(
        pl.kernel,
        out_type=jax.ShapeDtypeStruct((d_in * n_nodes,), jnp.float32),
        mesh=mesh,
        scratch_types=[
            pltpu.VMEM((words_per_w,), jnp.float32),
            pltpu.VMEM((words_per_w,), jnp.float32),
            pltpu.VMEM((_CHUNK,), jnp.int32),
            pltpu.VMEM((_CHUNK,), jnp.int32),
        ],
        compiler_params=pltpu.CompilerParams(needs_layout_passes=False),
    )
    def k(featT_hbm, src_hbm, dst_hbm, aggT_hbm, feat_v, agg_v, src_v, dst_v):
        wid = lax.axis_index("s") * _NC + lax.axis_index("c")
        base = wid * words_per_w
        pltpu.sync_copy(featT_hbm.at[pl.ds(base, words_per_w)], feat_v)

        def zero_body(i, _):
            agg_v[pl.ds(i * _L, _L)] = jnp.zeros((_L,), jnp.float32)
            return _

        lax.fori_loop(0, words_per_w // _L, zero_body, None)

        def chunk_body(kk, _):
            off = kk * _CHUNK
            pltpu.sync_copy(src_hbm.at[pl.ds(off, _CHUNK)], src_v)
            pltpu.sync_copy(dst_hbm.at[pl.ds(off, _CHUNK)], dst_v)

            def edge_body(i, _):
                s = src_v[pl.ds(i * _L, _L)]
                t = dst_v[pl.ds(i * _L, _L)]
                for c in range(cols_per_w):
                    vals = plsc.load_gather(feat_v, [s + c * n_nodes])
                    plsc.addupdate_scatter(agg_v, [t + c * n_nodes], vals)
                return _

            lax.fori_loop(0, _CHUNK // _L, edge_body, None)
            return _

        lax.fori_loop(0, n_chunks, chunk_body, None)
        pltpu.sync_copy(agg_v, aggT_hbm.at[pl.ds(base, words_per_w)])

    return k(featT_flat, src, dst)


def _tc_linear(agg, W, b2d, n_nodes, d_out):
    bn = 1000
    grid = (n_nodes // bn,)

    def body(agg_ref, w_ref, b_ref, out_ref):
        out_ref[...] = (
            lax.dot_general(
                agg_ref[...], w_ref[...], (((1,), (1,)), ((), ())),
                preferred_element_type=jnp.float32,
            )
            + b_ref[...]
        )

    return pl.pallas_call(
        body,
        out_shape=jax.ShapeDtypeStruct((n_nodes, d_out), jnp.float32),
        grid=grid,
        in_specs=[
            pl.BlockSpec((bn, agg.shape[1]), lambda i: (i, 0)),
            pl.BlockSpec(W.shape, lambda i: (0, 0)),
            pl.BlockSpec((1, d_out), lambda i: (0, 0)),
        ],
        out_specs=pl.BlockSpec((bn, d_out), lambda i: (i, 0)),
    )(agg, W, b2d)


def kernel(feat, edge_index, W, b):
    n_nodes, d_in = feat.shape
    d_out = W.shape[0]
    featT_flat = feat.T.reshape(-1)
    src = edge_index[0]
    dst = edge_index[1]
    aggT_flat = _sc_gather_scatter(featT_flat, src, dst, n_nodes, d_in)
    agg = aggT_flat.reshape(d_in, n_nodes).T
    return _tc_linear(agg, W, b.reshape(1, d_out), n_nodes, d_out)


# speedup vs baseline: 7.3673x; 2.3740x over previous
import functools

import jax
import jax.numpy as jnp
from jax import lax
from jax.experimental import pallas as pl
from jax.experimental.pallas import tpu as pltpu
from jax.experimental.pallas import tpu_sc as plsc

_NC = 2
_NS = 16
_L = 16
_NW = _NC * _NS

_CHUNK = 8000
_NBUF = 2


def _sc_gather_scatter(featT_flat, src, dst, n_nodes, d_in):
    cols_per_w = d_in // _NW
    words_per_w = cols_per_w * n_nodes
    n_edges = src.shape[0]
    n_chunks = n_edges // _CHUNK
    mesh = plsc.VectorSubcoreMesh(core_axis_name="c", subcore_axis_name="s")

    @functools.partial(
        pl.kernel,
        out_type=jax.ShapeDtypeStruct((d_in * n_nodes,), jnp.float32),
        mesh=mesh,
        scratch_types=[
            pltpu.VMEM((words_per_w,), jnp.float32),
            pltpu.VMEM((words_per_w,), jnp.float32),
            pltpu.VMEM((_CHUNK,), jnp.int32),
            pltpu.VMEM((_CHUNK,), jnp.int32),
            pltpu.VMEM((_CHUNK,), jnp.int32),
            pltpu.VMEM((_CHUNK,), jnp.int32),
            pltpu.SemaphoreType.DMA,
            pltpu.SemaphoreType.DMA,
        ],
        compiler_params=pltpu.CompilerParams(needs_layout_passes=False),
    )
    def k(featT_hbm, src_hbm, dst_hbm, aggT_hbm, feat_v, agg_v, src_v0, src_v1,
          dst_v0, dst_v1, sem0, sem1):
        sems = (sem0, sem1)
        src_bufs = (src_v0, src_v1)
        dst_bufs = (dst_v0, dst_v1)
        wid = lax.axis_index("s") * _NC + lax.axis_index("c")
        base = wid * words_per_w

        def start(b, ck):
            off = ck * _CHUNK
            pltpu.async_copy(src_hbm.at[pl.ds(off, _CHUNK)], src_bufs[b], sems[b])
            pltpu.async_copy(dst_hbm.at[pl.ds(off, _CHUNK)], dst_bufs[b], sems[b])

        def drain(b):
            pltpu.make_async_copy(src_hbm.at[pl.ds(0, _CHUNK)], src_bufs[b], sems[b]).wait()
            pltpu.make_async_copy(dst_hbm.at[pl.ds(0, _CHUNK)], dst_bufs[b], sems[b]).wait()

        for b in range(_NBUF):
            start(b, b)
        pltpu.sync_copy(featT_hbm.at[pl.ds(base, words_per_w)], feat_v)

        @plsc.parallel_loop(0, words_per_w // _L, unroll=8)
        def _zero(i):
            agg_v[pl.ds(i * _L, _L)] = jnp.zeros((_L,), jnp.float32)

        @pl.loop(0, n_chunks // _NBUF)
        def _outer(g):
            for b in range(_NBUF):
                ck = g * _NBUF + b
                drain(b)

                @plsc.parallel_loop(0, _CHUNK // _L, unroll=8)
                def _edges(i):
                    s = src_bufs[b][pl.ds(i * _L, _L)]
                    t = dst_bufs[b][pl.ds(i * _L, _L)]
                    for c in range(cols_per_w):
                        vals = plsc.load_gather(feat_v, [s + c * n_nodes])
                        plsc.addupdate_scatter(agg_v, [t + c * n_nodes], vals)

                nxt = ck + _NBUF

                @pl.when(nxt < n_chunks)
                def _():
                    start(b, nxt)

        pltpu.sync_copy(agg_v, aggT_hbm.at[pl.ds(base, words_per_w)])

    return k(featT_flat, src, dst)


def _tc_linear(agg, W, b2d, n_nodes, d_out):
    bn = 1000
    grid = (n_nodes // bn,)

    def body(agg_ref, w_ref, b_ref, out_ref):
        out_ref[...] = (
            lax.dot_general(
                agg_ref[...], w_ref[...], (((1,), (1,)), ((), ())),
                preferred_element_type=jnp.float32,
            )
            + b_ref[...]
        )

    return pl.pallas_call(
        body,
        out_shape=jax.ShapeDtypeStruct((n_nodes, d_out), jnp.float32),
        grid=grid,
        in_specs=[
            pl.BlockSpec((bn, agg.shape[1]), lambda i: (i, 0)),
            pl.BlockSpec(W.shape, lambda i: (0, 0)),
            pl.BlockSpec((1, d_out), lambda i: (0, 0)),
        ],
        out_specs=pl.BlockSpec((bn, d_out), lambda i: (i, 0)),
    )(agg, W, b2d)


def kernel(feat, edge_index, W, b):
    n_nodes, d_in = feat.shape
    d_out = W.shape[0]
    featT_flat = feat.T.reshape(-1)
    src = edge_index[0]
    dst = edge_index[1]
    aggT_flat = _sc_gather_scatter(featT_flat, src, dst, n_nodes, d_in)
    agg = aggT_flat.reshape(d_in, n_nodes).T
    return _tc_linear(agg, W, b.reshape(1, d_out), n_nodes, d_out)
